# baseline (speedup 1.0000x reference)
import functools

import jax
import jax.numpy as jnp
from jax import lax
from jax.experimental import pallas as pl
from jax.experimental.pallas import tpu as pltpu
from jax.experimental.pallas import tpu_sc as plsc

VOCAB = 1000000
EMBED = 64
NW = 32
FULLB = VOCAB // 128
TAILR = VOCAB - FULLB * 128


def _wid():
    return lax.axis_index("s") * 2 + lax.axis_index("c")


def _iota16():
    return lax.broadcasted_iota(jnp.int32, (16,), 0)


@functools.lru_cache(maxsize=None)
def _make_transpose():
    mesh = plsc.VectorSubcoreMesh(core_axis_name="c", subcore_axis_name="s")

    @functools.partial(
        pl.kernel,
        mesh=mesh,
        out_type=jax.ShapeDtypeStruct((VOCAB // 2, 128), jnp.float32),
        scratch_types=[
            pltpu.VMEM((2, 8, 8, 128), jnp.float32),
            pltpu.VMEM((2, 64, 128), jnp.float32),
            pltpu.VMEM((64, 128), jnp.float32),
            pltpu.SemaphoreType.DMA,
            pltpu.SemaphoreType.DMA,
            pltpu.SemaphoreType.DMA,
            pltpu.SemaphoreType.DMA,
        ],
        compiler_params=pltpu.CompilerParams(use_tc_tiling_on_sc=True, needs_layout_passes=False),
    )
    def transpose_kernel(wt_hbm, tail_hbm, tbl_hbm, tin, blk, tv,
                         is0, is1, os0, os1):
        w = _wid()
        isem = (is0, is1)
        osem = (os0, os1)
        base = w * (FULLB // NW) + jnp.minimum(w, FULLB % NW)
        nblk = FULLB // NW + jnp.where(w < FULLB % NW, 1, 0)
        j16 = _iota16()
        civ = [2 * k + j16 // 8 for k in range(4)]
        crv = j16 % 8

        def fire_in(g, b):
            rj = base + g
            for ci in range(8):
                pltpu.async_copy(
                    wt_hbm.at[pl.ds(8 * ci, 8), pl.ds(128 * rj, 128)],
                    tin.at[b, ci],
                    isem[b],
                )

        def drain_in(b):
            for ci in range(8):
                pltpu.make_async_copy(
                    wt_hbm.at[pl.ds(0, 8), pl.ds(0, 128)],
                    tin.at[b, ci],
                    isem[b],
                ).wait()

        def drain_out(b):
            pltpu.make_async_copy(
                tbl_hbm.at[pl.ds(0, 64)], blk.at[b], osem[b]
            ).wait()

        def do_transpose(bs, nrr):
            def rr_body(rr, c2):
                q = rr // 2
                col0 = lax.rem(rr, 2) * 64
                rrv = rr + j16 * 0
                for k in range(4):
                    v = plsc.load_gather(tin.at[bs], [civ[k], crv, rrv])
                    blk[bs, q, pl.ds(col0 + 16 * k, 16)] = v
                return c2

            lax.fori_loop(0, nrr, rr_body, 0)

        fire_in(0, 0)

        def body(g, carry):
            b = lax.rem(g, 2)
            for bs in range(2):
                @pl.when(b == bs)
                def _():
                    @pl.when(g + 1 < nblk)
                    def _():
                        fire_in(g + 1, 1 - bs)

                    drain_in(bs)

                    @pl.when(g >= 2)
                    def _():
                        drain_out(bs)

                    do_transpose(bs, 128)
                    pltpu.async_copy(
                        blk.at[bs],
                        tbl_hbm.at[pl.ds((base + g) * 64, 64)],
                        osem[bs],
                    )

            return carry

        lax.fori_loop(0, nblk, body, 0)
        for par in range(2):
            @pl.when(lax.rem(nblk, 2) == par)
            def _():
                drain_out(par)
                drain_out(1 - par)

        @pl.when(w == NW - 1)
        def _():
            pltpu.sync_copy(tail_hbm, tv)

            def q_body(q, c2):
                for half in range(2):
                    for i in range(4):
                        v = tv[2 * q + half, pl.ds(16 * i, 16)]
                        blk[0, q, pl.ds(64 * half + 16 * i, 16)] = v
                return c2

            lax.fori_loop(0, TAILR // 2, q_body, 0)
            pltpu.sync_copy(
                blk.at[0, pl.ds(0, TAILR // 2)],
                tbl_hbm.at[pl.ds(FULLB * 64, TAILR // 2)],
            )

    return transpose_kernel


@functools.lru_cache(maxsize=None)
def _make_gather():
    mesh = plsc.VectorSubcoreMesh(core_axis_name="c", subcore_axis_name="s")
    nblk = 6400 // NW

    @functools.partial(
        pl.kernel,
        mesh=mesh,
        out_type=jax.ShapeDtypeStruct((200, 8, 32, 8, 128), jnp.float32),
        scratch_types=[
            pltpu.VMEM((nblk * 128,), jnp.int32),
            pltpu.VMEM((nblk * 128,), jnp.int32),
            pltpu.VMEM((2, 128, 128), jnp.float32),
            pltpu.VMEM((2, 8, 8, 128), jnp.float32),
            pltpu.SemaphoreType.DMA,
            pltpu.SemaphoreType.DMA,
            pltpu.SemaphoreType.DMA,
            pltpu.SemaphoreType.DMA,
        ],
        compiler_params=pltpu.CompilerParams(use_tc_tiling_on_sc=False, needs_layout_passes=False),
    )
    def gather_kernel(tbl_hbm, idx_hbm, out_hbm, idxv, pidv, rows, blkt,
                      gs0, gs1, os0, os1):
        w = _wid()
        gsem = (gs0, gs1)
        osem = (os0, os1)
        base = w * nblk
        j16 = _iota16()

        pltpu.sync_copy(idx_hbm.at[pl.ds(base * 128, nblk * 128)], idxv)

        def shift_body(i, c2):
            x = idxv[pl.ds(i * 16, 16)]
            pidv[pl.ds(i * 16, 16)] = x >> 1
            return c2

        lax.fori_loop(0, nblk * 128 // 16, shift_body, 0)

        def fire_gather(g, b):
            pltpu.async_copy(
                tbl_hbm.at[pidv.at[pl.ds(g * 128, 128)]], rows.at[b], gsem[b]
            )

        def drain_gather(b):
            pltpu.make_async_copy(
                tbl_hbm.at[pl.ds(0, 128)], rows.at[b], gsem[b]
            ).wait()

        def drain_out(b):
            for ci in range(8):
                pltpu.make_async_copy(
                    tbl_hbm.at[pl.ds(0, 8)], blkt.at[b, ci], osem[b]
                ).wait()

        fire_gather(0, 0)

        def body(g, carry):
            b = lax.rem(g, 2)
            bid = base + g
            h = bid // 32
            bj = lax.rem(bid, 32)

            for bs in range(2):
                @pl.when(b == bs)
                def _():
                    @pl.when(g + 1 < nblk)
                    def _():
                        fire_gather(g + 1, 1 - bs)

                    drain_gather(bs)

                    @pl.when(g >= 2)
                    def _():
                        drain_out(bs)

                    def m_body(m, c2):
                        brv = m * 16 + j16
                        par = (idxv[pl.ds(g * 128 + m * 16, 16)] & 1) * 64

                        def c_body(c, c3):
                            v = plsc.load_gather(rows.at[bs], [brv, par + c])
                            blkt[bs, c // 8, lax.rem(c, 8),
                                 pl.ds(m * 16, 16)] = v
                            return c3

                        lax.fori_loop(0, 64, c_body, 0, unroll=4)
                        return c2

                    lax.fori_loop(0, 8, m_body, 0)

                    for ci in range(8):
                        pltpu.async_copy(
                            blkt.at[bs, ci], out_hbm.at[h, ci, bj], osem[bs]
                        )

            return carry

        lax.fori_loop(0, nblk, body, 0)
        drain_out(0)
        drain_out(1)

    return gather_kernel


def kernel(indices, weight):
    wt_t = weight.T
    tailp = jnp.pad(weight[FULLB * 128:], ((0, 0), (0, 128 - EMBED)))
    tblp = _make_transpose()(wt_t, tailp)
    idx_flat = indices.T.reshape(-1)
    out5 = _make_gather()(tblp, idx_flat)
    return out5.transpose(2, 4, 0, 1, 3).reshape(4096, 200, 64)


# speedup vs baseline: 1.1158x; 1.1158x over previous
import functools

import jax
import jax.numpy as jnp
from jax import lax
from jax.experimental import pallas as pl
from jax.experimental.pallas import tpu as pltpu
from jax.experimental.pallas import tpu_sc as plsc

VOCAB = 1000000
EMBED = 64
NW = 32
FULLB = VOCAB // 128
TAILR = VOCAB - FULLB * 128


def _wid():
    return lax.axis_index("s") * 2 + lax.axis_index("c")


def _iota16():
    return lax.broadcasted_iota(jnp.int32, (16,), 0)


@functools.lru_cache(maxsize=None)
def _make_transpose():
    mesh = plsc.VectorSubcoreMesh(core_axis_name="c", subcore_axis_name="s")

    @functools.partial(
        pl.kernel,
        mesh=mesh,
        out_type=jax.ShapeDtypeStruct((VOCAB * EMBED,), jnp.float32),
        scratch_types=[
            pltpu.VMEM((2, 8, 8, 128), jnp.float32),
            pltpu.VMEM((8192,), jnp.float32),
            pltpu.VMEM((8192,), jnp.float32),
            pltpu.VMEM((64, 128), jnp.float32),
            pltpu.SemaphoreType.DMA,
            pltpu.SemaphoreType.DMA,
            pltpu.SemaphoreType.DMA,
            pltpu.SemaphoreType.DMA,
        ],
        compiler_params=pltpu.CompilerParams(use_tc_tiling_on_sc=True, needs_layout_passes=False, disable_bounds_checks=True),
    )
    def transpose_kernel(wt_hbm, tail_hbm, tbl_hbm, tin, blk0, blk1, tv,
                         is0, is1, os0, os1):
        blk = (blk0, blk1)
        w = _wid()
        isem = (is0, is1)
        osem = (os0, os1)
        base = w * (FULLB // NW) + jnp.minimum(w, FULLB % NW)
        nblk = FULLB // NW + jnp.where(w < FULLB % NW, 1, 0)
        j16 = _iota16()
        j64 = j16 * 64

        def fire_in(g, b):
            rj = base + g
            for ci in range(8):
                pltpu.async_copy(
                    wt_hbm.at[pl.ds(8 * ci, 8), pl.ds(128 * rj, 128)],
                    tin.at[b, ci],
                    isem[b],
                )

        def drain_in(b):
            for ci in range(8):
                pltpu.make_async_copy(
                    wt_hbm.at[pl.ds(0, 8), pl.ds(0, 128)],
                    tin.at[b, ci],
                    isem[b],
                ).wait()

        def drain_out(b):
            pltpu.make_async_copy(
                tbl_hbm.at[pl.ds(0, 8192)], blk[b], osem[b]
            ).wait()

        def do_transpose(bs):
            def m_body(m, c2):
                m1024 = m * 1024
                for ci in range(8):
                    for cr in range(8):
                        v = tin[bs, ci, cr, pl.ds(16 * m, 16)]
                        plsc.store_scatter(
                            blk[bs], [j64 + (m1024 + ci * 8 + cr)], v
                        )
                return c2

            lax.fori_loop(0, 8, m_body, 0)

        fire_in(0, 0)

        def body(g, carry):
            b = lax.rem(g, 2)
            for bs in range(2):
                @pl.when(b == bs)
                def _():
                    @pl.when(g + 1 < nblk)
                    def _():
                        fire_in(g + 1, 1 - bs)

                    drain_in(bs)

                    @pl.when(g >= 2)
                    def _():
                        drain_out(bs)

                    do_transpose(bs)
                    pltpu.async_copy(
                        blk[bs],
                        tbl_hbm.at[pl.ds((base + g) * 8192, 8192)],
                        osem[bs],
                    )

            return carry

        lax.fori_loop(0, nblk, body, 0)
        for par in range(2):
            @pl.when(lax.rem(nblk, 2) == par)
            def _():
                drain_out(par)
                drain_out(1 - par)

        @pl.when(w == NW - 1)
        def _():
            pltpu.sync_copy(tail_hbm, tv)

            def q_body(rr, c2):
                for i in range(4):
                    v = tv[rr, pl.ds(16 * i, 16)]
                    blk0[pl.ds(rr * 64 + 16 * i, 16)] = v
                return c2

            lax.fori_loop(0, TAILR, q_body, 0)
            pltpu.sync_copy(
                blk0.at[pl.ds(0, TAILR * 64)],
                tbl_hbm.at[pl.ds(FULLB * 8192, TAILR * 64)],
            )

    return transpose_kernel


@functools.lru_cache(maxsize=None)
def _make_gather():
    mesh = plsc.VectorSubcoreMesh(core_axis_name="c", subcore_axis_name="s")
    nblk = 6400 // NW

    @functools.partial(
        pl.kernel,
        mesh=mesh,
        out_type=jax.ShapeDtypeStruct((200, 8, 32, 8, 128), jnp.float32),
        scratch_types=[
            pltpu.VMEM((nblk * 128,), jnp.int32),
            pltpu.VMEM((nblk * 128,), jnp.int32),
            pltpu.VMEM((2, 128, 128), jnp.float32),
            pltpu.VMEM((2, 8, 8, 128), jnp.float32),
            pltpu.SemaphoreType.DMA,
            pltpu.SemaphoreType.DMA,
            pltpu.SemaphoreType.DMA,
            pltpu.SemaphoreType.DMA,
        ],
        compiler_params=pltpu.CompilerParams(use_tc_tiling_on_sc=False, needs_layout_passes=False, disable_bounds_checks=True),
    )
    def gather_kernel(tbl_hbm, idx_hbm, out_hbm, idxv, pidv, rows, blkt,
                      gs0, gs1, os0, os1):
        w = _wid()
        gsem = (gs0, gs1)
        osem = (os0, os1)
        base = w * nblk
        j16 = _iota16()

        pltpu.sync_copy(idx_hbm.at[pl.ds(base * 128, nblk * 128)], idxv)

        def shift_body(i, c2):
            x = idxv[pl.ds(i * 16, 16)]
            pidv[pl.ds(i * 16, 16)] = x >> 1
            return c2

        lax.fori_loop(0, nblk * 128 // 16, shift_body, 0)

        def fire_gather(g, b):
            pltpu.async_copy(
                tbl_hbm.at[pidv.at[pl.ds(g * 128, 128)]], rows.at[b], gsem[b]
            )

        def drain_gather(b):
            pltpu.make_async_copy(
                tbl_hbm.at[pl.ds(0, 128)], rows.at[b], gsem[b]
            ).wait()

        def drain_out(b):
            for ci in range(8):
                pltpu.make_async_copy(
                    tbl_hbm.at[pl.ds(0, 8)], blkt.at[b, ci], osem[b]
                ).wait()

        fire_gather(0, 0)

        def body(g, carry):
            b = lax.rem(g, 2)
            bid = base + g
            h = bid // 32
            bj = lax.rem(bid, 32)

            for bs in range(2):
                @pl.when(b == bs)
                def _():
                    @pl.when(g + 1 < nblk)
                    def _():
                        fire_gather(g + 1, 1 - bs)

                    drain_gather(bs)

                    @pl.when(g >= 2)
                    def _():
                        drain_out(bs)

                    def m_body(m, c2):
                        brv = m * 16 + j16
                        par = (idxv[pl.ds(g * 128 + m * 16, 16)] & 1) * 64

                        def c_body(c, c3):
                            v = plsc.load_gather(rows.at[bs], [brv, par + c])
                            blkt[bs, c // 8, lax.rem(c, 8),
                                 pl.ds(m * 16, 16)] = v
                            return c3

                        lax.fori_loop(0, 64, c_body, 0, unroll=4)
                        return c2

                    lax.fori_loop(0, 8, m_body, 0)

                    for ci in range(8):
                        pltpu.async_copy(
                            blkt.at[bs, ci], out_hbm.at[h, ci, bj], osem[bs]
                        )

            return carry

        lax.fori_loop(0, nblk, body, 0)
        drain_out(0)
        drain_out(1)

    return gather_kernel


def kernel(indices, weight):
    wt_t = weight.T
    tailp = jnp.pad(weight[FULLB * 128:], ((0, 0), (0, 128 - EMBED)))
    tblp = _make_transpose()(wt_t, tailp).reshape(VOCAB // 2, 128)
    idx_flat = indices.T.reshape(-1)
    out5 = _make_gather()(tblp, idx_flat)
    return out5.transpose(2, 4, 0, 1, 3).reshape(4096, 200, 64)


# speedup vs baseline: 1.4900x; 1.3353x over previous
import functools

import jax
import jax.numpy as jnp
from jax import lax
from jax.experimental import pallas as pl
from jax.experimental.pallas import tpu as pltpu
from jax.experimental.pallas import tpu_sc as plsc

VOCAB = 1000000
EMBED = 64
NW = 32
FULLB = VOCAB // 128
TAILR = VOCAB - FULLB * 128


def _wid():
    return lax.axis_index("s") * 2 + lax.axis_index("c")


def _iota16():
    return lax.broadcasted_iota(jnp.int32, (16,), 0)


@functools.lru_cache(maxsize=None)
def _make_transpose():
    mesh = plsc.VectorSubcoreMesh(core_axis_name="c", subcore_axis_name="s")

    @functools.partial(
        pl.kernel,
        mesh=mesh,
        out_type=jax.ShapeDtypeStruct((VOCAB * EMBED,), jnp.float32),
        scratch_types=[
            pltpu.VMEM((2, 8, 8, 128), jnp.float32),
            pltpu.VMEM((8192,), jnp.float32),
            pltpu.VMEM((8192,), jnp.float32),
            pltpu.VMEM((64, 128), jnp.float32),
            pltpu.SemaphoreType.DMA,
            pltpu.SemaphoreType.DMA,
            pltpu.SemaphoreType.DMA,
            pltpu.SemaphoreType.DMA,
        ],
        compiler_params=pltpu.CompilerParams(use_tc_tiling_on_sc=True, needs_layout_passes=False, disable_bounds_checks=True),
    )
    def transpose_kernel(wt_hbm, tail_hbm, tbl_hbm, tin, blk0, blk1, tv,
                         is0, is1, os0, os1):
        blk = (blk0, blk1)
        w = _wid()
        isem = (is0, is1)
        osem = (os0, os1)
        base = w * (FULLB // NW) + jnp.minimum(w, FULLB % NW)
        nblk = FULLB // NW + jnp.where(w < FULLB % NW, 1, 0)
        j16 = _iota16()
        j64 = j16 * 64

        def fire_in(g, b):
            rj = base + g
            for ci in range(8):
                pltpu.async_copy(
                    wt_hbm.at[pl.ds(8 * ci, 8), pl.ds(128 * rj, 128)],
                    tin.at[b, ci],
                    isem[b],
                )

        def drain_in(b):
            for ci in range(8):
                pltpu.make_async_copy(
                    wt_hbm.at[pl.ds(0, 8), pl.ds(0, 128)],
                    tin.at[b, ci],
                    isem[b],
                ).wait()

        def drain_out(b):
            pltpu.make_async_copy(
                tbl_hbm.at[pl.ds(0, 8192)], blk[b], osem[b]
            ).wait()

        def do_transpose(bs):
            @plsc.parallel_loop(0, 8)
            def m_body(m):
                m1024 = m * 1024
                for ci in range(8):
                    for cr in range(8):
                        v = tin[bs, ci, cr, pl.ds(16 * m, 16)]
                        plsc.store_scatter(
                            blk[bs], [j64 + (m1024 + ci * 8 + cr)], v
                        )

        fire_in(0, 0)

        def body(g, carry):
            b = lax.rem(g, 2)
            for bs in range(2):
                @pl.when(b == bs)
                def _():
                    @pl.when(g + 1 < nblk)
                    def _():
                        fire_in(g + 1, 1 - bs)

                    drain_in(bs)

                    @pl.when(g >= 2)
                    def _():
                        drain_out(bs)

                    do_transpose(bs)
                    pltpu.async_copy(
                        blk[bs],
                        tbl_hbm.at[pl.ds((base + g) * 8192, 8192)],
                        osem[bs],
                    )

            return carry

        lax.fori_loop(0, nblk, body, 0)
        for par in range(2):
            @pl.when(lax.rem(nblk, 2) == par)
            def _():
                drain_out(par)
                drain_out(1 - par)

        @pl.when(w == NW - 1)
        def _():
            pltpu.sync_copy(tail_hbm, tv)

            def q_body(rr, c2):
                for i in range(4):
                    v = tv[rr, pl.ds(16 * i, 16)]
                    blk0[pl.ds(rr * 64 + 16 * i, 16)] = v
                return c2

            lax.fori_loop(0, TAILR, q_body, 0)
            pltpu.sync_copy(
                blk0.at[pl.ds(0, TAILR * 64)],
                tbl_hbm.at[pl.ds(FULLB * 8192, TAILR * 64)],
            )

    return transpose_kernel


@functools.lru_cache(maxsize=None)
def _make_gather():
    mesh = plsc.VectorSubcoreMesh(core_axis_name="c", subcore_axis_name="s")
    nblk = 6400 // NW

    @functools.partial(
        pl.kernel,
        mesh=mesh,
        out_type=jax.ShapeDtypeStruct((200, 8, 32, 8, 128), jnp.float32),
        scratch_types=[
            pltpu.VMEM((nblk * 128,), jnp.int32),
            pltpu.VMEM((nblk * 128,), jnp.int32),
            pltpu.VMEM((2, 128, 128), jnp.float32),
            pltpu.VMEM((2, 8, 8, 128), jnp.float32),
            pltpu.SemaphoreType.DMA,
            pltpu.SemaphoreType.DMA,
            pltpu.SemaphoreType.DMA,
            pltpu.SemaphoreType.DMA,
        ],
        compiler_params=pltpu.CompilerParams(use_tc_tiling_on_sc=False, needs_layout_passes=False, disable_bounds_checks=True),
    )
    def gather_kernel(tbl_hbm, idx_hbm, out_hbm, idxv, pidv, rows, blkt,
                      gs0, gs1, os0, os1):
        w = _wid()
        gsem = (gs0, gs1)
        osem = (os0, os1)
        base = w * nblk
        j16 = _iota16()

        pltpu.sync_copy(idx_hbm.at[pl.ds(base * 128, nblk * 128)], idxv)

        def shift_body(i, c2):
            x = idxv[pl.ds(i * 16, 16)]
            pidv[pl.ds(i * 16, 16)] = x >> 1
            return c2

        lax.fori_loop(0, nblk * 128 // 16, shift_body, 0)

        def fire_gather(g, b):
            pltpu.async_copy(
                tbl_hbm.at[pidv.at[pl.ds(g * 128, 128)]], rows.at[b], gsem[b]
            )

        def drain_gather(b):
            pltpu.make_async_copy(
                tbl_hbm.at[pl.ds(0, 128)], rows.at[b], gsem[b]
            ).wait()

        def drain_out(b):
            for ci in range(8):
                pltpu.make_async_copy(
                    tbl_hbm.at[pl.ds(0, 8)], blkt.at[b, ci], osem[b]
                ).wait()

        fire_gather(0, 0)

        def body(g, carry):
            b = lax.rem(g, 2)
            bid = base + g
            h = bid // 32
            bj = lax.rem(bid, 32)

            for bs in range(2):
                @pl.when(b == bs)
                def _():
                    @pl.when(g + 1 < nblk)
                    def _():
                        fire_gather(g + 1, 1 - bs)

                    drain_gather(bs)

                    @pl.when(g >= 2)
                    def _():
                        drain_out(bs)

                    @plsc.parallel_loop(0, 8)
                    def m_body(m):
                        brv = m * 16 + j16
                        par = (idxv[pl.ds(g * 128 + m * 16, 16)] & 1) * 64
                        for ci in range(8):
                            for cr in range(8):
                                v = plsc.load_gather(
                                    rows.at[bs], [brv, par + (ci * 8 + cr)]
                                )
                                blkt[bs, ci, cr, pl.ds(m * 16, 16)] = v

                    for ci in range(8):
                        pltpu.async_copy(
                            blkt.at[bs, ci], out_hbm.at[h, ci, bj], osem[bs]
                        )

            return carry

        lax.fori_loop(0, nblk, body, 0)
        drain_out(0)
        drain_out(1)

    return gather_kernel


def kernel(indices, weight):
    wt_t = weight.T
    tailp = jnp.pad(weight[FULLB * 128:], ((0, 0), (0, 128 - EMBED)))
    tblp = _make_transpose()(wt_t, tailp).reshape(VOCAB // 2, 128)
    idx_flat = indices.T.reshape(-1)
    out5 = _make_gather()(tblp, idx_flat)
    return out5.transpose(2, 4, 0, 1, 3).reshape(4096, 200, 64)


# speedup vs baseline: 1.4953x; 1.0035x over previous
import functools

import jax
import jax.numpy as jnp
from jax import lax
from jax.experimental import pallas as pl
from jax.experimental.pallas import tpu as pltpu
from jax.experimental.pallas import tpu_sc as plsc

VOCAB = 1000000
EMBED = 64
NW = 32
FULLB = VOCAB // 128
TAILR = VOCAB - FULLB * 128


def _wid():
    return lax.axis_index("s") * 2 + lax.axis_index("c")


def _iota16():
    return lax.broadcasted_iota(jnp.int32, (16,), 0)


@functools.lru_cache(maxsize=None)
def _make_transpose():
    mesh = plsc.VectorSubcoreMesh(core_axis_name="c", subcore_axis_name="s")

    @functools.partial(
        pl.kernel,
        mesh=mesh,
        out_type=jax.ShapeDtypeStruct((VOCAB // 2, 128), jnp.float32),
        scratch_types=[
            pltpu.VMEM((2, 8, 8, 128), jnp.float32),
            pltpu.VMEM((64, 129), jnp.float32),
            pltpu.VMEM((64, 129), jnp.float32),
            pltpu.VMEM((64, 128), jnp.float32),
            pltpu.SemaphoreType.DMA,
            pltpu.SemaphoreType.DMA,
            pltpu.SemaphoreType.DMA,
            pltpu.SemaphoreType.DMA,
        ],
        compiler_params=pltpu.CompilerParams(use_tc_tiling_on_sc=True, needs_layout_passes=False, disable_bounds_checks=True),
    )
    def transpose_kernel(wt_hbm, tail_hbm, tbl_hbm, tin, blk0, blk1, tv,
                         is0, is1, os0, os1):
        blk = (blk0, blk1)
        w = _wid()
        isem = (is0, is1)
        osem = (os0, os1)
        base = w * (FULLB // NW) + jnp.minimum(w, FULLB % NW)
        nblk = FULLB // NW + jnp.where(w < FULLB % NW, 1, 0)
        j16 = _iota16()
        jq = j16 // 2
        jcol = (j16 & 1) * 64

        def fire_in(g, b):
            rj = base + g
            for ci in range(8):
                pltpu.async_copy(
                    wt_hbm.at[pl.ds(8 * ci, 8), pl.ds(128 * rj, 128)],
                    tin.at[b, ci],
                    isem[b],
                )

        def drain_in(b):
            for ci in range(8):
                pltpu.make_async_copy(
                    wt_hbm.at[pl.ds(0, 8), pl.ds(0, 128)],
                    tin.at[b, ci],
                    isem[b],
                ).wait()

        def blk2d(b):
            return blk[b].at[:, pl.ds(0, 128)]

        def drain_out(b):
            pltpu.make_async_copy(
                tbl_hbm.at[pl.ds(0, 64)], blk2d(b), osem[b]
            ).wait()

        def do_transpose(bs):
            @plsc.parallel_loop(0, 8)
            def m_body(m):
                qv = jq + 8 * m
                for ci in range(8):
                    for cr in range(8):
                        v = tin[bs, ci, cr, pl.ds(16 * m, 16)]
                        plsc.store_scatter(
                            blk[bs], [qv, jcol + (ci * 8 + cr)], v
                        )

        fire_in(0, 0)

        def body(g, carry):
            b = lax.rem(g, 2)
            for bs in range(2):
                @pl.when(b == bs)
                def _():
                    @pl.when(g + 1 < nblk)
                    def _():
                        fire_in(g + 1, 1 - bs)

                    drain_in(bs)

                    @pl.when(g >= 2)
                    def _():
                        drain_out(bs)

                    do_transpose(bs)
                    pltpu.async_copy(
                        blk2d(bs),
                        tbl_hbm.at[pl.ds((base + g) * 64, 64)],
                        osem[bs],
                    )

            return carry

        lax.fori_loop(0, nblk, body, 0)
        for par in range(2):
            @pl.when(lax.rem(nblk, 2) == par)
            def _():
                drain_out(par)
                drain_out(1 - par)

        @pl.when(w == NW - 1)
        def _():
            pltpu.sync_copy(tail_hbm, tv)

            def q_body(rr, c2):
                q = rr // 2
                col0 = lax.rem(rr, 2) * 64
                for i in range(4):
                    v = tv[rr, pl.ds(16 * i, 16)]
                    blk0[q, pl.ds(col0 + 16 * i, 16)] = v
                return c2

            lax.fori_loop(0, TAILR, q_body, 0)
            pltpu.sync_copy(
                blk0.at[pl.ds(0, TAILR // 2), pl.ds(0, 128)],
                tbl_hbm.at[pl.ds(FULLB * 64, TAILR // 2)],
            )

    return transpose_kernel


@functools.lru_cache(maxsize=None)
def _make_gather():
    mesh = plsc.VectorSubcoreMesh(core_axis_name="c", subcore_axis_name="s")
    nblk = 6400 // NW

    @functools.partial(
        pl.kernel,
        mesh=mesh,
        out_type=jax.ShapeDtypeStruct((200, 8, 32, 8, 128), jnp.float32),
        scratch_types=[
            pltpu.VMEM((nblk * 128,), jnp.int32),
            pltpu.VMEM((nblk * 128,), jnp.int32),
            pltpu.VMEM((2, 128, 128), jnp.float32),
            pltpu.VMEM((2, 8, 8, 128), jnp.float32),
            pltpu.SemaphoreType.DMA,
            pltpu.SemaphoreType.DMA,
            pltpu.SemaphoreType.DMA,
            pltpu.SemaphoreType.DMA,
        ],
        compiler_params=pltpu.CompilerParams(use_tc_tiling_on_sc=False, needs_layout_passes=False, disable_bounds_checks=True),
    )
    def gather_kernel(tbl_hbm, idx_hbm, out_hbm, idxv, pidv, rows, blkt,
                      gs0, gs1, os0, os1):
        w = _wid()
        gsem = (gs0, gs1)
        osem = (os0, os1)
        base = w * nblk
        j16 = _iota16()

        pltpu.sync_copy(idx_hbm.at[pl.ds(base * 128, nblk * 128)], idxv)

        def shift_body(i, c2):
            x = idxv[pl.ds(i * 16, 16)]
            pidv[pl.ds(i * 16, 16)] = x >> 1
            return c2

        lax.fori_loop(0, nblk * 128 // 16, shift_body, 0)

        def fire_gather(g, b):
            pltpu.async_copy(
                tbl_hbm.at[pidv.at[pl.ds(g * 128, 128)]], rows.at[b], gsem[b]
            )

        def drain_gather(b):
            pltpu.make_async_copy(
                tbl_hbm.at[pl.ds(0, 128)], rows.at[b], gsem[b]
            ).wait()

        def drain_out(b):
            for ci in range(8):
                pltpu.make_async_copy(
                    tbl_hbm.at[pl.ds(0, 8)], blkt.at[b, ci], osem[b]
                ).wait()

        fire_gather(0, 0)

        def body(g, carry):
            b = lax.rem(g, 2)
            bid = base + g
            h = bid // 32
            bj = lax.rem(bid, 32)

            for bs in range(2):
                @pl.when(b == bs)
                def _():
                    @pl.when(g + 1 < nblk)
                    def _():
                        fire_gather(g + 1, 1 - bs)

                    drain_gather(bs)

                    @pl.when(g >= 2)
                    def _():
                        drain_out(bs)

                    @plsc.parallel_loop(0, 8)
                    def m_body(m):
                        brv = m * 16 + j16
                        par = (idxv[pl.ds(g * 128 + m * 16, 16)] & 1) * 64
                        for ci in range(8):
                            for cr in range(8):
                                v = plsc.load_gather(
                                    rows.at[bs], [brv, par + (ci * 8 + cr)]
                                )
                                blkt[bs, ci, cr, pl.ds(m * 16, 16)] = v

                    for ci in range(8):
                        pltpu.async_copy(
                            blkt.at[bs, ci], out_hbm.at[h, ci, bj], osem[bs]
                        )

            return carry

        lax.fori_loop(0, nblk, body, 0)
        drain_out(0)
        drain_out(1)

    return gather_kernel


def kernel(indices, weight):
    wt_t = weight.T
    tailp = jnp.pad(weight[FULLB * 128:], ((0, 0), (0, 128 - EMBED)))
    tblp = _make_transpose()(wt_t, tailp)
    idx_flat = indices.T.reshape(-1)
    out5 = _make_gather()(tblp, idx_flat)
    return out5.transpose(2, 4, 0, 1, 3).reshape(4096, 200, 64)


# speedup vs baseline: 2.2273x; 1.4896x over previous
import functools

import jax
import jax.numpy as jnp
from jax import lax
from jax.experimental import pallas as pl
from jax.experimental.pallas import tpu as pltpu
from jax.experimental.pallas import tpu_sc as plsc

EMBED = 64


@functools.lru_cache(maxsize=None)
def _make_gather(batch, hist):
    info = plsc.get_sparse_core_info()
    nc, ns = info.num_cores, info.num_subcores
    nw = nc * ns
    bpw = batch // nw
    assert bpw % 2 == 0
    mesh = plsc.VectorSubcoreMesh(core_axis_name="c", subcore_axis_name="s")

    @functools.partial(
        pl.kernel,
        mesh=mesh,
        out_type=jax.ShapeDtypeStruct((batch, hist, EMBED), jnp.float32),
        scratch_types=[
            pltpu.VMEM((bpw, hist), jnp.int32),
            pltpu.VMEM((2, hist, EMBED), jnp.float32),
            pltpu.SemaphoreType.DMA,
            pltpu.SemaphoreType.DMA,
        ],
        compiler_params=pltpu.CompilerParams(use_tc_tiling_on_sc=False),
    )
    def gather_kernel(table_hbm, idx_hbm, out_hbm, idx_v, rows_v, gsem0, gsem1):
        wid = lax.axis_index("s") * nc + lax.axis_index("c")
        gsem = (gsem0, gsem1)
        b0 = wid * bpw

        pltpu.sync_copy(idx_hbm.at[pl.ds(b0, bpw)], idx_v)

        def fire_gather(t, b):
            pltpu.async_copy(
                table_hbm.at[idx_v.at[t]], rows_v.at[b], gsem[b]
            )

        def drain_gather(b):
            pltpu.make_async_copy(
                out_hbm.at[0], rows_v.at[b], gsem[b]
            ).wait()

        fire_gather(0, 0)
        fire_gather(1, 1)

        def outer(t2, carry):
            for b in range(2):
                t = t2 * 2 + b
                drain_gather(b)
                wcp = pltpu.make_async_copy(
                    rows_v.at[b], out_hbm.at[b0 + t], gsem[b]
                )
                wcp.start()
                wcp.wait()

                @pl.when(t2 < bpw // 2 - 1)
                def _():
                    fire_gather(t + 2, b)

            return carry

        lax.fori_loop(0, bpw // 2, outer, 0)

    return gather_kernel


def kernel(indices, weight):
    batch, hist = indices.shape
    return _make_gather(batch, hist)(weight, indices)
